# baseline (speedup 1.0000x reference)
import functools

import jax
import jax.numpy as jnp
from jax import lax
from jax.experimental import pallas as pl
from jax.experimental.pallas import tpu as pltpu
from jax.experimental.pallas import tpu_sc as plsc

_TILE = 8192
_K = 1024
_D = 32
_CW = 0.25


def _vq_tc_kernel(x_ref, e_ref, idx_ref, part_ref):
    x = x_ref[...]
    e = e_ref[...]
    x2 = jnp.sum(x * x, axis=1, keepdims=True)
    e2 = jnp.sum(e * e, axis=1, keepdims=True).T
    xe2 = lax.dot_general(2.0 * x, e, (((1,), (1,)), ((), ())),
                          preferred_element_type=jnp.float32)
    dist = (x2 - xe2) + e2
    idx = jnp.argmin(dist, axis=1).astype(jnp.int32)
    idx_ref[0, 0, :] = idx
    part_ref[0, 0, 0] = jnp.sum(jnp.min(dist, axis=1))


def _make_sc_gather(n):
    info = plsc.get_sparse_core_info()
    nw = info.num_cores * info.num_subcores
    b_per_w = n // nw
    mesh = plsc.VectorSubcoreMesh(core_axis_name="c", subcore_axis_name="s")

    @functools.partial(
        pl.kernel, mesh=mesh,
        out_type=jax.ShapeDtypeStruct((n, 128), jnp.float32),
        scratch_types=[
            pltpu.VMEM((b_per_w,), jnp.int32),
            pltpu.VMEM((b_per_w, 128), jnp.float32),
            pltpu.SemaphoreType.DMA,
        ],
    )
    def sc_gather(table_hbm, idx_hbm, out_hbm, idx_v, rows_v, sem):
        wid = lax.axis_index("s") * info.num_cores + lax.axis_index("c")
        base = wid * b_per_w
        pltpu.sync_copy(idx_hbm.at[pl.ds(base, b_per_w)], idx_v)
        pltpu.async_copy(table_hbm.at[idx_v], rows_v, sem).wait()
        pltpu.sync_copy(rows_v, out_hbm.at[pl.ds(base, b_per_w)])

    return sc_gather


def kernel(x, embed):
    B, T, D = x.shape
    xf = x.reshape(-1, D)
    n = xf.shape[0]
    g = n // _TILE
    idx3, parts = pl.pallas_call(
        _vq_tc_kernel,
        grid=(g,),
        in_specs=[pl.BlockSpec((_TILE, D), lambda i: (i, 0)),
                  pl.BlockSpec((_K, D), lambda i: (0, 0))],
        out_specs=[pl.BlockSpec((1, 1, _TILE), lambda i: (i, 0, 0)),
                   pl.BlockSpec((1, 1, 1), lambda i: (i, 0, 0),
                                memory_space=pltpu.SMEM)],
        out_shape=[jax.ShapeDtypeStruct((g, 1, _TILE), jnp.int32),
                   jax.ShapeDtypeStruct((g, 1, 1), jnp.float32)],
        compiler_params=pltpu.CompilerParams(
            dimension_semantics=("parallel",)),
    )(xf, embed)
    table = jnp.pad(embed, ((0, 0), (0, 128 - D)))
    q = _make_sc_gather(n)(table, idx3.reshape(n))[:, :D]
    loss = (1.0 + _CW) * jnp.sum(parts) / (n * D)
    return q.reshape(B, T, D), idx3.reshape(B, T), loss


# speedup vs baseline: 1.0845x; 1.0845x over previous
import functools

import jax
import jax.numpy as jnp
from jax import lax
from jax.experimental import pallas as pl
from jax.experimental.pallas import tpu as pltpu
from jax.experimental.pallas import tpu_sc as plsc

_TILE = 8192
_K = 1024
_D = 32
_CW = 0.25


def _vq_tc_kernel(x_ref, e_ref, idx_ref, part_ref):
    x = x_ref[...]
    e = e_ref[...]
    x2 = jnp.sum(x * x, axis=1, keepdims=True)
    e2 = jnp.sum(e * e, axis=1, keepdims=True).T
    xe2 = lax.dot_general(2.0 * x, e, (((1,), (1,)), ((), ())),
                          preferred_element_type=jnp.float32)
    dist = (x2 - xe2) + e2
    idx = jnp.argmin(dist, axis=1).astype(jnp.int32)
    idx_ref[0, 0, :] = idx
    part_ref[0, 0, 0] = jnp.sum(jnp.min(dist, axis=1))


def _make_sc_gather(n):
    info = plsc.get_sparse_core_info()
    nw = info.num_cores * info.num_subcores
    b_per_w = n // nw
    mesh = plsc.VectorSubcoreMesh(core_axis_name="c", subcore_axis_name="s")

    @functools.partial(
        pl.kernel, mesh=mesh,
        out_type=jax.ShapeDtypeStruct((n, _D), jnp.float32),
        scratch_types=[
            pltpu.VMEM((b_per_w,), jnp.int32),
            pltpu.VMEM((b_per_w, _D), jnp.float32),
            pltpu.SemaphoreType.DMA,
        ],
        compiler_params=pltpu.CompilerParams(use_tc_tiling_on_sc=False),
    )
    def sc_gather(table_hbm, idx_hbm, out_hbm, idx_v, rows_v, sem):
        wid = lax.axis_index("s") * info.num_cores + lax.axis_index("c")
        base = wid * b_per_w
        pltpu.sync_copy(idx_hbm.at[pl.ds(base, b_per_w)], idx_v)
        pltpu.async_copy(table_hbm.at[idx_v], rows_v, sem).wait()
        pltpu.sync_copy(rows_v, out_hbm.at[pl.ds(base, b_per_w)])

    return sc_gather


def kernel(x, embed):
    B, T, D = x.shape
    xf = x.reshape(-1, D)
    n = xf.shape[0]
    g = n // _TILE
    idx3, parts = pl.pallas_call(
        _vq_tc_kernel,
        grid=(g,),
        in_specs=[pl.BlockSpec((_TILE, D), lambda i: (i, 0)),
                  pl.BlockSpec((_K, D), lambda i: (0, 0))],
        out_specs=[pl.BlockSpec((1, 1, _TILE), lambda i: (i, 0, 0)),
                   pl.BlockSpec((1, 1, 1), lambda i: (i, 0, 0),
                                memory_space=pltpu.SMEM)],
        out_shape=[jax.ShapeDtypeStruct((g, 1, _TILE), jnp.int32),
                   jax.ShapeDtypeStruct((g, 1, 1), jnp.float32)],
        compiler_params=pltpu.CompilerParams(
            dimension_semantics=("parallel",)),
    )(xf, embed)
    q = _make_sc_gather(n)(embed, idx3.reshape(n))
    loss = (1.0 + _CW) * jnp.sum(parts) / (n * D)
    return q.reshape(B, T, D), idx3.reshape(B, T), loss


# speedup vs baseline: 1.2296x; 1.1337x over previous
import functools

import jax
import jax.numpy as jnp
from jax import lax
from jax.experimental import pallas as pl
from jax.experimental.pallas import tpu as pltpu
from jax.experimental.pallas import tpu_sc as plsc

_TILE = 8192
_K = 1024
_D = 32
_CW = 0.25


def _vq_tc_kernel(x_ref, e_ref, idx_ref, part_ref):
    x = x_ref[...]
    e = e_ref[...]
    x2 = jnp.sum(x * x, axis=1, keepdims=True)
    e2 = jnp.sum(e * e, axis=1, keepdims=True).T
    xe2 = lax.dot_general(2.0 * x, e, (((1,), (1,)), ((), ())),
                          preferred_element_type=jnp.float32)
    dist = (x2 - xe2) + e2
    m = dist[:, :128]
    c = jnp.zeros(m.shape, jnp.int32)
    for b in range(1, _K // 128):
        db = dist[:, b * 128:(b + 1) * 128]
        lt = db < m
        m = jnp.where(lt, db, m)
        c = jnp.where(lt, b, c)
    min_d = jnp.min(m, axis=1, keepdims=True)
    lane = jax.lax.broadcasted_iota(jnp.int32, m.shape, 1)
    gidx = c * 128 + lane
    idx = jnp.min(jnp.where(m == min_d, gidx, _K), axis=1)
    idx_ref[0, 0, :] = idx
    part_ref[0, 0, 0] = jnp.sum(min_d)


def _make_sc_gather(n):
    info = plsc.get_sparse_core_info()
    nw = info.num_cores * info.num_subcores
    b_per_w = n // nw
    mesh = plsc.VectorSubcoreMesh(core_axis_name="c", subcore_axis_name="s")

    @functools.partial(
        pl.kernel, mesh=mesh,
        out_type=jax.ShapeDtypeStruct((n, _D), jnp.float32),
        scratch_types=[
            pltpu.VMEM((b_per_w,), jnp.int32),
            pltpu.VMEM((b_per_w, _D), jnp.float32),
            pltpu.SemaphoreType.DMA,
        ],
        compiler_params=pltpu.CompilerParams(use_tc_tiling_on_sc=False),
    )
    def sc_gather(table_hbm, idx_hbm, out_hbm, idx_v, rows_v, sem):
        wid = lax.axis_index("s") * info.num_cores + lax.axis_index("c")
        base = wid * b_per_w
        pltpu.sync_copy(idx_hbm.at[pl.ds(base, b_per_w)], idx_v)
        pltpu.async_copy(table_hbm.at[idx_v], rows_v, sem).wait()
        pltpu.sync_copy(rows_v, out_hbm.at[pl.ds(base, b_per_w)])

    return sc_gather


def kernel(x, embed):
    B, T, D = x.shape
    xf = x.reshape(-1, D)
    n = xf.shape[0]
    g = n // _TILE
    idx3, parts = pl.pallas_call(
        _vq_tc_kernel,
        grid=(g,),
        in_specs=[pl.BlockSpec((_TILE, D), lambda i: (i, 0)),
                  pl.BlockSpec((_K, D), lambda i: (0, 0))],
        out_specs=[pl.BlockSpec((1, 1, _TILE), lambda i: (i, 0, 0)),
                   pl.BlockSpec((1, 1, 1), lambda i: (i, 0, 0),
                                memory_space=pltpu.SMEM)],
        out_shape=[jax.ShapeDtypeStruct((g, 1, _TILE), jnp.int32),
                   jax.ShapeDtypeStruct((g, 1, 1), jnp.float32)],
        compiler_params=pltpu.CompilerParams(
            dimension_semantics=("parallel",)),
    )(xf, embed)
    q = _make_sc_gather(n)(embed, idx3.reshape(n))
    loss = (1.0 + _CW) * jnp.sum(parts) / (n * D)
    return q.reshape(B, T, D), idx3.reshape(B, T), loss


# speedup vs baseline: 1.4544x; 1.1829x over previous
import jax
import jax.numpy as jnp
from jax import lax
from jax.experimental import pallas as pl
from jax.experimental.pallas import tpu as pltpu

_TILE = 8192
_K = 1024
_D = 32
_CW = 0.25


def _vq_kernel(x_ref, e_ref, e3_ref, q_ref, idx_ref, part_ref):
    x = x_ref[...]
    e = e_ref[...]
    x2 = jnp.sum(x * x, axis=1, keepdims=True)
    e2 = jnp.sum(e * e, axis=1, keepdims=True).T
    xe2 = lax.dot_general(2.0 * x, e, (((1,), (1,)), ((), ())),
                          preferred_element_type=jnp.float32)
    dist = (x2 - xe2) + e2
    m = dist[:, :128]
    c = jnp.zeros(m.shape, jnp.int32)
    for b in range(1, _K // 128):
        db = dist[:, b * 128:(b + 1) * 128]
        lt = db < m
        m = jnp.where(lt, db, m)
        c = jnp.where(lt, b, c)
    min_d = jnp.min(m, axis=1, keepdims=True)
    lane = lax.broadcasted_iota(jnp.int32, m.shape, 1)
    gidx = c * 128 + lane
    idx = jnp.min(jnp.where(m == min_d, gidx, _K), axis=1)
    lane_k = lax.broadcasted_iota(jnp.int32, dist.shape, 1)
    onehot = (lane_k == idx[:, None]).astype(jnp.bfloat16)
    q3 = lax.dot_general(onehot, e3_ref[...], (((1,), (0,)), ((), ())),
                         preferred_element_type=jnp.float32)
    q = (q3[:, :_D] + q3[:, _D:2 * _D]) + q3[:, 2 * _D:]
    q_ref[...] = x + (q - x)
    idx_ref[0, 0, :] = idx
    part_ref[0, 0, 0] = jnp.sum(min_d)


def kernel(x, embed):
    B, T, D = x.shape
    xf = x.reshape(-1, D)
    n = xf.shape[0]
    g = n // _TILE
    hi = embed.astype(jnp.bfloat16).astype(jnp.float32)
    r1 = embed - hi
    mid = r1.astype(jnp.bfloat16).astype(jnp.float32)
    lo = r1 - mid
    e3 = jnp.concatenate([hi, mid, lo], axis=1).astype(jnp.bfloat16)
    q, idx3, parts = pl.pallas_call(
        _vq_kernel,
        grid=(g,),
        in_specs=[pl.BlockSpec((_TILE, D), lambda i: (i, 0)),
                  pl.BlockSpec((_K, D), lambda i: (0, 0)),
                  pl.BlockSpec((_K, 3 * D), lambda i: (0, 0))],
        out_specs=[pl.BlockSpec((_TILE, D), lambda i: (i, 0)),
                   pl.BlockSpec((1, 1, _TILE), lambda i: (i, 0, 0)),
                   pl.BlockSpec((1, 1, 1), lambda i: (i, 0, 0),
                                memory_space=pltpu.SMEM)],
        out_shape=[jax.ShapeDtypeStruct((n, D), jnp.float32),
                   jax.ShapeDtypeStruct((g, 1, _TILE), jnp.int32),
                   jax.ShapeDtypeStruct((g, 1, 1), jnp.float32)],
        compiler_params=pltpu.CompilerParams(
            dimension_semantics=("parallel",)),
    )(xf, embed, e3)
    loss = (1.0 + _CW) * jnp.sum(parts) / (n * D)
    return q.reshape(B, T, D), idx3.reshape(B, T), loss
